# baseline (speedup 1.0000x reference)
import functools

import jax
import jax.numpy as jnp
import numpy as np
from jax import lax
from jax.experimental import pallas as pl
from jax.experimental.pallas import tpu as pltpu
from jax.experimental.pallas import tpu_sc as plsc

_NC = 2
_NS = 16
_NW = _NC * _NS
_L = 16


def kernel(x_cat, tables):
    B, F = x_cat.shape
    _, V, D = tables.shape
    total = B * F
    C = 128
    n_rows = total // C
    rows_w = n_rows // _NW
    pat_rows = 13
    wide = 128
    rpw = wide // D

    tab128 = tables.reshape(-1, wide)
    idx2d = x_cat.astype(jnp.int32).reshape(n_rows, C)
    pat_np = (np.arange(pat_rows * C).reshape(pat_rows, C) % F) * (V // rpw)
    phi = jnp.asarray(pat_np.astype(np.int32))

    mesh = plsc.VectorSubcoreMesh(core_axis_name="c", subcore_axis_name="s")

    @functools.partial(
        pl.kernel,
        mesh=mesh,
        out_type=jax.ShapeDtypeStruct((total * D // wide, wide), jnp.float32),
        compiler_params=pltpu.CompilerParams(needs_layout_passes=False),
        scratch_types=[
            pltpu.VMEM((rows_w, C), jnp.int32),
            pltpu.VMEM((rows_w, C), jnp.int32),
            pltpu.VMEM((pat_rows, C), jnp.int32),
            pltpu.VMEM((C, wide), jnp.float32),
            pltpu.VMEM((D, wide), jnp.float32),
            pltpu.SemaphoreType.DMA,
        ],
    )
    def emb(tab_hbm, idx_hbm, phi_hbm, out_hbm, hi_v, lo_v, phi_v, s_v, o_v, sem):
        wid = lax.axis_index("s") * _NC + lax.axis_index("c")
        row0 = wid * rows_w

        pltpu.sync_copy(idx_hbm.at[pl.ds(row0, rows_w)], hi_v)
        pltpu.sync_copy(phi_hbm, phi_v)

        def prep(r, carry):
            pr = lax.rem(r, pat_rows)
            for v in range(C // _L):
                sl = pl.ds(v * _L, _L)
                raw = hi_v[r, sl]
                lo_v[r, sl] = lax.bitwise_and(raw, 7)
                hi_v[r, sl] = lax.shift_right_logical(raw, 3) + phi_v[pr, sl]
            return carry

        lax.fori_loop(0, rows_w, prep, None)

        iota = lax.iota(jnp.int32, _L)

        def chunk(r, carry):
            pltpu.async_copy(tab_hbm.at[hi_v.at[r]], s_v, sem).wait()
            for g in range(C // _L):
                rows = g * _L + iota
                colb = lo_v[r, pl.ds(g * _L, _L)] * D
                orow = 2 * g + lax.shift_right_logical(iota, 3)
                ocol0 = lax.bitwise_and(iota, 7) * D
                for c in range(D):
                    t = plsc.load_gather(s_v, [rows, colb + c])
                    plsc.store_scatter(o_v, [orow, ocol0 + c], t)
            pltpu.sync_copy(o_v, out_hbm.at[pl.ds((row0 + r) * D, D)])
            return carry

        lax.fori_loop(0, rows_w, chunk, None)

    out = emb(tab128, idx2d, phi)
    return out.reshape(B, F * D)
